# baseline (speedup 1.0000x reference)
import dataclasses
import functools

import jax
import jax.numpy as jnp
from jax import lax
from jax.experimental import pallas as pl
from jax.experimental.pallas import tpu as pltpu
from jax.experimental.pallas import tpu_sc as plsc

B, S = 64, 512
N = B * S
D = 128
P = 512
EPS = 1e-12
NC, NS = 2, 16
NW = NC * NS
TOK_PER_W = N // NW
W = 128
NWIN = TOK_PER_W // W
NBUF = 2
LANES = 16
NCH = D // LANES
MAGIC = 0x5F3759DF


def _sc_fused(word, pos_ext, wids, comb, gamma, beta):
    mesh = plsc.VectorSubcoreMesh(core_axis_name="c", subcore_axis_name="s")
    cp = pltpu.CompilerParams()
    if "needs_layout_passes" in pltpu.CompilerParams.__dataclass_fields__:
        cp = dataclasses.replace(cp, needs_layout_passes=False)

    @functools.partial(
        pl.kernel,
        mesh=mesh,
        compiler_params=cp,
        out_type=jax.ShapeDtypeStruct((N, D), jnp.float32),
        scratch_types=[
            pltpu.VMEM((TOK_PER_W,), jnp.int32),
            pltpu.VMEM((TOK_PER_W,), jnp.int32),
            pltpu.VMEM((D,), jnp.float32),
            pltpu.VMEM((D,), jnp.float32),
            pltpu.VMEM((NBUF, W, D), jnp.float32),
            pltpu.VMEM((NBUF, W, D), jnp.float32),
            pltpu.VMEM((NBUF, W, D), jnp.float32),
            pltpu.VMEM_SHARED((2 * P, D), jnp.float32),
            pltpu.SemaphoreType.DMA,
            pltpu.SemaphoreType.DMA,
            pltpu.SemaphoreType.DMA,
            pltpu.SemaphoreType.DMA,
            pltpu.SemaphoreType.DMA,
            pltpu.SemaphoreType.DMA,
        ],
    )
    def k(word_hbm, pose_hbm, wid_hbm, cmb_hbm, gam_hbm, bet_hbm, out_hbm,
          widx_v, cidx_v, gam_v, bet_v,
          wrow_v, prow_v, srow_v, pos_spm,
          gw0, gw1, gp0, gp1, os0, os1):
        gsemw = (gw0, gw1)
        gsemp = (gp0, gp1)
        osem = (os0, os1)
        sid = lax.axis_index("s")
        w_id = sid * NC + lax.axis_index("c")
        base = w_id * TOK_PER_W

        @pl.when(sid == 0)
        def _stage():
            pltpu.sync_copy(pose_hbm, pos_spm)

        pltpu.sync_copy(wid_hbm.at[pl.ds(base, TOK_PER_W)], widx_v)
        pltpu.sync_copy(cmb_hbm.at[pl.ds(base, TOK_PER_W)], cidx_v)
        pltpu.sync_copy(gam_hbm, gam_v)
        pltpu.sync_copy(bet_hbm, bet_v)

        gc, bc = [], []
        for j in range(NCH):
            cs = pl.ds(j * LANES, LANES)
            gc.append(gam_v.at[cs][...])
            bc.append(bet_v.at[cs][...])

        def issue_gathers(g, b):
            isl = pl.ds(g * W, W)
            pltpu.async_copy(word_hbm.at[widx_v.at[isl]], wrow_v.at[b],
                             gsemw[b])
            pltpu.async_copy(pos_spm.at[cidx_v.at[isl]], prow_v.at[b],
                             gsemp[b])

        def wait_gathers(b):
            pltpu.make_async_copy(word_hbm.at[pl.ds(0, W)], wrow_v.at[b],
                                  gsemw[b]).wait()
            pltpu.make_async_copy(word_hbm.at[pl.ds(0, W)], prow_v.at[b],
                                  gsemp[b]).wait()

        def wait_out(b):
            pltpu.make_async_copy(srow_v.at[b], out_hbm.at[pl.ds(base, W)],
                                  osem[b]).wait()

        plsc.subcore_barrier()

        for b in range(NBUF):
            issue_gathers(b, b)

        @pl.loop(0, NWIN, step=NBUF)
        def _ring(g0):
            for b in range(NBUF):
                g = g0 + b
                @pl.when(g0 > 0)
                def _():
                    wait_out(b)

                wait_gathers(b)
                wb = wrow_v.at[b]
                pb = prow_v.at[b]
                sb = srow_v.at[b]

                @plsc.parallel_loop(0, W, unroll=2)
                def _row(r):
                    e = []
                    for j in range(NCH):
                        cs = pl.ds(j * LANES, LANES)
                        e.append(wb.at[r, cs][...] + pb.at[r, cs][...])
                    a0 = (e[0] + e[1]) + (e[2] + e[3])
                    a1 = (e[4] + e[5]) + (e[6] + e[7])
                    s1 = jnp.sum(a0 + a1)
                    q0 = (e[0] * e[0] + e[1] * e[1]) + (e[2] * e[2]
                                                        + e[3] * e[3])
                    q1 = (e[4] * e[4] + e[5] * e[5]) + (e[6] * e[6]
                                                        + e[7] * e[7])
                    s2 = jnp.sum(q0 + q1)
                    mean = s1 * (1.0 / D)
                    var = s2 * (1.0 / D) - mean * mean
                    x = var + EPS
                    xi = lax.bitcast_convert_type(x, jnp.int32)
                    yi = jnp.int32(MAGIC) - lax.shift_right_logical(xi, 1)
                    y = lax.bitcast_convert_type(yi, jnp.float32)
                    y = y * (1.5 - 0.5 * x * y * y)
                    y = y * (1.5 - 0.5 * x * y * y)
                    y = y * (1.5 - 0.5 * x * y * y)
                    mf = jnp.full((LANES,), mean, jnp.float32)
                    yf = jnp.full((LANES,), y, jnp.float32)
                    for j in range(NCH):
                        cs = pl.ds(j * LANES, LANES)
                        sb.at[r, cs][...] = ((e[j] - mf) * yf * gc[j]
                                             + bc[j])

                pltpu.async_copy(sb, out_hbm.at[pl.ds(base + g * W, W)],
                                 osem[b])

                @pl.when(g + NBUF < NWIN)
                def _():
                    issue_gathers(g + NBUF, b)

        for b in range(NBUF):
            wait_out(b)

    return k(word, pos_ext, wids, comb, gamma, beta)


def kernel(input_ids, position_ids, token_type_ids, word_embeddings,
           position_embeddings, token_type_embeddings, gamma, beta):
    wids = input_ids.reshape(-1).astype(jnp.int32)
    comb = (position_ids.reshape(-1).astype(jnp.int32)
            + P * token_type_ids.reshape(-1).astype(jnp.int32))
    pos_ext = jnp.concatenate([
        position_embeddings + token_type_embeddings[0],
        position_embeddings + token_type_embeddings[1],
    ])
    out = _sc_fused(word_embeddings, pos_ext, wids, comb, gamma, beta)
    return out.reshape(B, S, D)


# speedup vs baseline: 1.0336x; 1.0336x over previous
import dataclasses
import functools

import jax
import jax.numpy as jnp
from jax import lax
from jax.experimental import pallas as pl
from jax.experimental.pallas import tpu as pltpu
from jax.experimental.pallas import tpu_sc as plsc

B, S = 64, 512
N = B * S
D = 128
P = 512
EPS = 1e-12
NC, NS = 2, 16
NW = NC * NS
TOK_PER_W = N // NW
W = 128
NWIN = TOK_PER_W // W
NBUF = 2
LANES = 16
NCH = D // LANES
MAGIC = 0x5F3759DF


def _sc_fused(word, pos_ext, ids2, gamma, beta):
    mesh = plsc.VectorSubcoreMesh(core_axis_name="c", subcore_axis_name="s")
    cp = pltpu.CompilerParams()
    if "needs_layout_passes" in pltpu.CompilerParams.__dataclass_fields__:
        cp = dataclasses.replace(cp, needs_layout_passes=False)

    @functools.partial(
        pl.kernel,
        mesh=mesh,
        compiler_params=cp,
        out_type=jax.ShapeDtypeStruct((N, D), jnp.float32),
        scratch_types=[
            pltpu.VMEM((2 * TOK_PER_W,), jnp.int32),
            pltpu.VMEM((D,), jnp.float32),
            pltpu.VMEM((D,), jnp.float32),
            pltpu.VMEM((NBUF, W, D), jnp.float32),
            pltpu.VMEM((NBUF, W, D), jnp.float32),
            pltpu.VMEM((NBUF, W, D), jnp.float32),
            pltpu.VMEM_SHARED((2 * P, D), jnp.float32),
            pltpu.SemaphoreType.DMA,
            pltpu.SemaphoreType.DMA,
            pltpu.SemaphoreType.DMA,
            pltpu.SemaphoreType.DMA,
            pltpu.SemaphoreType.DMA,
            pltpu.SemaphoreType.DMA,
        ],
    )
    def k(word_hbm, pose_hbm, ids_hbm, gam_hbm, bet_hbm, out_hbm,
          idx_v, gam_v, bet_v,
          wrow_v, prow_v, srow_v, pos_spm,
          gw0, gw1, gp0, gp1, os0, os1):
        gsemw = (gw0, gw1)
        gsemp = (gp0, gp1)
        osem = (os0, os1)
        sid = lax.axis_index("s")
        w_id = sid * NC + lax.axis_index("c")
        base = w_id * TOK_PER_W

        @pl.when(sid == 0)
        def _stage():
            pltpu.sync_copy(pose_hbm, pos_spm)

        pltpu.sync_copy(ids_hbm.at[pl.ds(w_id * 2 * TOK_PER_W,
                                         2 * TOK_PER_W)], idx_v)

        def issue_word(g, b):
            pltpu.async_copy(word_hbm.at[idx_v.at[pl.ds(g * W, W)]],
                             wrow_v.at[b], gsemw[b])

        def issue_pos(g, b):
            pltpu.async_copy(
                pos_spm.at[idx_v.at[pl.ds(TOK_PER_W + g * W, W)]],
                prow_v.at[b], gsemp[b])

        def issue_gathers(g, b):
            issue_word(g, b)
            issue_pos(g, b)

        def wait_gathers(b):
            pltpu.make_async_copy(word_hbm.at[pl.ds(0, W)], wrow_v.at[b],
                                  gsemw[b]).wait()
            pltpu.make_async_copy(word_hbm.at[pl.ds(0, W)], prow_v.at[b],
                                  gsemp[b]).wait()

        def wait_out(b):
            pltpu.make_async_copy(srow_v.at[b], out_hbm.at[pl.ds(base, W)],
                                  osem[b]).wait()

        for b in range(NBUF):
            issue_word(b, b)

        plsc.subcore_barrier()
        for b in range(NBUF):
            issue_pos(b, b)

        pltpu.sync_copy(gam_hbm, gam_v)
        pltpu.sync_copy(bet_hbm, bet_v)
        gc, bc = [], []
        for j in range(NCH):
            cs = pl.ds(j * LANES, LANES)
            gc.append(gam_v.at[cs][...])
            bc.append(bet_v.at[cs][...])

        @pl.loop(0, NWIN, step=NBUF)
        def _ring(g0):
            for b in range(NBUF):
                g = g0 + b
                @pl.when(g0 > 0)
                def _():
                    wait_out(b)

                wait_gathers(b)
                wb = wrow_v.at[b]
                pb = prow_v.at[b]
                sb = srow_v.at[b]

                @plsc.parallel_loop(0, W, unroll=2)
                def _row(r):
                    e = []
                    for j in range(NCH):
                        cs = pl.ds(j * LANES, LANES)
                        e.append(wb.at[r, cs][...] + pb.at[r, cs][...])
                    a0 = (e[0] + e[1]) + (e[2] + e[3])
                    a1 = (e[4] + e[5]) + (e[6] + e[7])
                    s1 = jnp.sum(a0 + a1)
                    q0 = (e[0] * e[0] + e[1] * e[1]) + (e[2] * e[2]
                                                        + e[3] * e[3])
                    q1 = (e[4] * e[4] + e[5] * e[5]) + (e[6] * e[6]
                                                        + e[7] * e[7])
                    s2 = jnp.sum(q0 + q1)
                    mean = s1 * (1.0 / D)
                    var = s2 * (1.0 / D) - mean * mean
                    x = var + EPS
                    xi = lax.bitcast_convert_type(x, jnp.int32)
                    yi = jnp.int32(MAGIC) - lax.shift_right_logical(xi, 1)
                    y = lax.bitcast_convert_type(yi, jnp.float32)
                    y = y * (1.5 - 0.5 * x * y * y)
                    y = y * (1.5 - 0.5 * x * y * y)
                    y = y * (1.5 - 0.5 * x * y * y)
                    mf = jnp.full((LANES,), mean, jnp.float32)
                    yf = jnp.full((LANES,), y, jnp.float32)
                    for j in range(NCH):
                        cs = pl.ds(j * LANES, LANES)
                        sb.at[r, cs][...] = ((e[j] - mf) * yf * gc[j]
                                             + bc[j])

                pltpu.async_copy(sb, out_hbm.at[pl.ds(base + g * W, W)],
                                 osem[b])

                @pl.when(g + NBUF < NWIN)
                def _():
                    issue_gathers(g + NBUF, b)

        for b in range(NBUF):
            wait_out(b)

    return k(word, pos_ext, ids2, gamma, beta)


def kernel(input_ids, position_ids, token_type_ids, word_embeddings,
           position_embeddings, token_type_embeddings, gamma, beta):
    wids = input_ids.reshape(-1).astype(jnp.int32)
    comb = (position_ids.reshape(-1).astype(jnp.int32)
            + P * token_type_ids.reshape(-1).astype(jnp.int32))
    ids2 = jnp.stack([wids.reshape(NW, TOK_PER_W),
                      comb.reshape(NW, TOK_PER_W)], axis=1).reshape(-1)
    pos_ext = jnp.concatenate([
        position_embeddings + token_type_embeddings[0],
        position_embeddings + token_type_embeddings[1],
    ])
    out = _sc_fused(word_embeddings, pos_ext, ids2, gamma, beta)
    return out.reshape(B, S, D)
